# Initial kernel scaffold; baseline (speedup 1.0000x reference)
import jax
import jax.numpy as jnp
from jax.experimental import pallas as pl


def kernel(node_features, edge_index, edge_weights, Wp1, bp1, Wu1, bu1, Wp2, bp2, Wu2, bu2):
    raise NotImplementedError("write your pallas kernel here")


# speedup vs baseline: 2.6432x; 2.6432x over previous
import dataclasses
import functools

import jax
import jax.numpy as jnp
from jax import lax
from jax.experimental import pallas as pl
from jax.experimental.pallas import tpu as pltpu
from jax.experimental.pallas import tpu_sc as plsc

def _sc_params():
    cp = pltpu.CompilerParams()
    if "needs_layout_passes" in pltpu.CompilerParams.__dataclass_fields__:
        cp = dataclasses.replace(cp, needs_layout_passes=False)
    return cp


_NC = 2
_NS = 16
_L = 16
_CH = 128


def _make_edge_weight_kernel(e_pad, n_pad):
    chunks = e_pad // _CH
    cpt = chunks // _NS
    cpw = chunks // (_NC * _NS)
    stripe = n_pad // _NS

    mesh = plsc.VectorSubcoreMesh(core_axis_name="c", subcore_axis_name="s")

    @functools.partial(
        pl.kernel,
        out_type=jax.ShapeDtypeStruct((e_pad,), jnp.float32),
        mesh=mesh,
        scratch_types=[
            pltpu.VMEM((1, _CH), jnp.int32),
            pltpu.VMEM((_CH,), jnp.float32),
            pltpu.VMEM((stripe,), jnp.float32),
            pltpu.VMEM((n_pad,), jnp.float32),
            pltpu.VMEM((_CH,), jnp.int32),
            pltpu.VMEM((_CH,), jnp.float32),
            pltpu.VMEM((_CH,), jnp.float32),
            pltpu.VMEM_SHARED((n_pad,), jnp.float32),
        ],
        compiler_params=_sc_params(),
    )
    def ker(dst_hbm, ew_hbm, ew2_hbm, dstr, ones_v, zer_v, cntl, dch, ech, och,
            cnt_sh):
        c = lax.axis_index("c")
        s = lax.axis_index("s")

        for j in range(_CH // _L):
            ones_v[pl.ds(j * _L, _L)] = jnp.full((_L,), 1.0, jnp.float32)
        for j in range(stripe // _L):
            zer_v[pl.ds(j * _L, _L)] = jnp.zeros((_L,), jnp.float32)
        pltpu.sync_copy(zer_v, cnt_sh.at[pl.ds(s * stripe, stripe)])
        plsc.subcore_barrier()

        @pl.loop(0, cpt)
        def _(i):
            ch = s * cpt + i
            pltpu.sync_copy(dst_hbm.at[pl.ds(ch * _CH, _CH)], dstr.at[0])
            pltpu.sync_copy(ones_v, cnt_sh.at[dstr.at[0]], add=True)

        plsc.subcore_barrier()
        pltpu.sync_copy(cnt_sh, cntl)

        w = s * _NC + c

        @pl.loop(0, cpw)
        def _(i):
            ch = w * cpw + i
            pltpu.sync_copy(dst_hbm.at[pl.ds(ch * _CH, _CH)], dch)
            pltpu.sync_copy(ew_hbm.at[pl.ds(ch * _CH, _CH)], ech)
            for j in range(_CH // _L):
                sl = pl.ds(j * _L, _L)
                cv = plsc.load_gather(cntl, [dch[sl]])
                och[sl] = ech[sl] / jnp.maximum(cv, 1.0)
            pltpu.sync_copy(och, ew2_hbm.at[pl.ds(ch * _CH, _CH)])

    return ker


def _make_agg_kernel(e_pad, n_pad, hh):
    chunks = e_pad // _CH
    cpt = chunks // _NS
    stripe = n_pad // _NS

    mesh = plsc.VectorSubcoreMesh(core_axis_name="c", subcore_axis_name="s")

    @functools.partial(
        pl.kernel,
        out_type=jax.ShapeDtypeStruct((_NC, n_pad, hh), jnp.float32),
        mesh=mesh,
        scratch_types=[
            pltpu.VMEM((cpt * _CH,), jnp.int32),
            pltpu.VMEM((cpt, _CH), jnp.int32),
            pltpu.VMEM((cpt * _CH,), jnp.float32),
            pltpu.VMEM((_CH, hh), jnp.float32),
            pltpu.VMEM_SHARED((n_pad, hh), jnp.float32),
        ],
        compiler_params=_sc_params(),
    )
    def ker(y_hbm, src_hbm, dst2_hbm, ew2_hbm, agg_hbm, srcb, dstb, ewb, rows,
            agg_sh):
        c = lax.axis_index("c")
        s = lax.axis_index("s")

        @pl.loop(0, _CH)
        def _(r):
            for j in range(hh // _L):
                rows[r, pl.ds(j * _L, _L)] = jnp.zeros((_L,), jnp.float32)

        @pl.loop(0, stripe // _CH)
        def _(t):
            pltpu.sync_copy(rows, agg_sh.at[pl.ds(s * stripe + t * _CH, _CH)])

        base = s * cpt * _CH
        pltpu.sync_copy(src_hbm.at[pl.ds(base, cpt * _CH)], srcb)
        pltpu.sync_copy(dst2_hbm.at[pl.ds(s * cpt, cpt)], dstb)
        pltpu.sync_copy(ew2_hbm.at[pl.ds(base, cpt * _CH)], ewb)
        off = c * n_pad

        @pl.loop(0, cpt * _CH // _L)
        def _(k):
            sl = pl.ds(k * _L, _L)
            srcb[sl] = srcb[sl] + off

        plsc.subcore_barrier()

        @pl.loop(0, cpt)
        def _(i):
            pltpu.sync_copy(y_hbm.at[srcb.at[pl.ds(i * _CH, _CH)]], rows)

            @pl.loop(0, _CH // _L)
            def _(g):
                w16 = ewb[pl.ds(i * _CH + g * _L, _L)]
                for k in range(_L):
                    e = g * _L + k
                    wgt = w16[k]
                    for j in range(hh // _L):
                        sl = pl.ds(j * _L, _L)
                        rows[e, sl] = rows[e, sl] * wgt

            pltpu.sync_copy(rows, agg_sh.at[dstb.at[i]], add=True)

        plsc.subcore_barrier()
        pltpu.sync_copy(agg_sh.at[pl.ds(s * stripe, stripe)],
                        agg_hbm.at[c].at[pl.ds(s * stripe, stripe)])

    return ker


def _prep_body(x_ref, w_ref, b_ref, o_ref):
    acc = jnp.dot(x_ref[...], w_ref[...],
                  preferred_element_type=jnp.float32,
                  precision=lax.Precision.HIGHEST)
    o_ref[...] = jnp.maximum(acc + b_ref[...], 0.0)


def _prepare(x_p, wp, bp, bn):
    n_pad, d = x_p.shape
    h = wp.shape[1]
    halves = h // 128
    nblk = n_pad // bn
    return pl.pallas_call(
        _prep_body,
        grid=(nblk, halves),
        in_specs=[
            pl.BlockSpec((bn, d), lambda i, hh: (i, 0)),
            pl.BlockSpec((d, 128), lambda i, hh: (0, hh)),
            pl.BlockSpec((1, 128), lambda i, hh: (0, hh)),
        ],
        out_specs=pl.BlockSpec((bn, 128), lambda i, hh, _n=nblk: (hh * _n + i, 0)),
        out_shape=jax.ShapeDtypeStruct((halves * n_pad, 128), jnp.float32),
    )(x_p, wp, bp.reshape(1, h))


def _upd_body(x_ref, a0_ref, a1_ref, wx_ref, w0_ref, w1_ref, b_ref, o_ref):
    acc = jnp.dot(x_ref[...], wx_ref[...],
                  preferred_element_type=jnp.float32,
                  precision=lax.Precision.HIGHEST)
    acc += jnp.dot(a0_ref[0], w0_ref[...],
                   preferred_element_type=jnp.float32,
                   precision=lax.Precision.HIGHEST)
    acc += jnp.dot(a1_ref[0], w1_ref[...],
                   preferred_element_type=jnp.float32,
                   precision=lax.Precision.HIGHEST)
    o_ref[...] = jnp.maximum(acc + b_ref[...], 0.0)


def _update(x_p, agg, wu, bu, bn):
    n_pad, d = x_p.shape
    h = wu.shape[1]
    hh = agg.shape[2]
    wx = wu[:d]
    w0 = wu[d:d + hh]
    w1 = wu[d + hh:]
    return pl.pallas_call(
        _upd_body,
        grid=(n_pad // bn,),
        in_specs=[
            pl.BlockSpec((bn, d), lambda i: (i, 0)),
            pl.BlockSpec((1, bn, hh), lambda i: (0, i, 0)),
            pl.BlockSpec((1, bn, hh), lambda i: (1, i, 0)),
            pl.BlockSpec((d, h), lambda i: (0, 0)),
            pl.BlockSpec((hh, h), lambda i: (0, 0)),
            pl.BlockSpec((hh, h), lambda i: (0, 0)),
            pl.BlockSpec((1, h), lambda i: (0, 0)),
        ],
        out_specs=pl.BlockSpec((bn, h), lambda i: (i, 0)),
        out_shape=jax.ShapeDtypeStruct((n_pad, h), jnp.float32),
    )(x_p, agg, agg, wx, w0, w1, bu.reshape(1, h))


def kernel(node_features, edge_index, edge_weights, Wp1, bp1, Wu1, bu1,
           Wp2, bp2, Wu2, bu2):
    n, d = node_features.shape
    h = Wp1.shape[1]
    e = edge_weights.shape[0]
    hh = h // _NC

    bn = 2048
    n_pad = -(-max(n + 1, 1) // bn) * bn
    e_pad = -(-e // (_CH * _NC * _NS)) * (_CH * _NC * _NS)

    src = edge_index[0]
    dst = edge_index[1]
    pad_e = e_pad - e
    src_p = jnp.concatenate([src, jnp.zeros((pad_e,), jnp.int32)])
    dst_p = jnp.concatenate([dst, jnp.full((pad_e,), n, jnp.int32)])
    ew_p = jnp.concatenate([edge_weights, jnp.zeros((pad_e,), jnp.float32)])
    dst2 = dst_p.reshape(e_pad // _CH, _CH)
    x_p = jnp.zeros((n_pad, d), jnp.float32).at[:n].set(node_features)

    ew2 = _make_edge_weight_kernel(e_pad, n_pad)(dst_p, ew_p)

    y1 = _prepare(x_p, Wp1, bp1, bn)
    agg1 = _make_agg_kernel(e_pad, n_pad, hh)(y1, src_p, dst2, ew2)
    e1 = _update(x_p, agg1, Wu1, bu1, bn)

    y2 = _prepare(e1, Wp2, bp2, bn)
    agg2 = _make_agg_kernel(e_pad, n_pad, hh)(y2, src_p, dst2, ew2)
    e2 = _update(e1, agg2, Wu2, bu2, bn)

    return e2[:n]


# speedup vs baseline: 3.1960x; 1.2091x over previous
import dataclasses
import functools

import jax
import jax.numpy as jnp
from jax import lax
from jax.experimental import pallas as pl
from jax.experimental.pallas import tpu as pltpu
from jax.experimental.pallas import tpu_sc as plsc

def _sc_params():
    cp = pltpu.CompilerParams()
    if "needs_layout_passes" in pltpu.CompilerParams.__dataclass_fields__:
        cp = dataclasses.replace(cp, needs_layout_passes=False)
    return cp


_NC = 2
_NS = 16
_L = 16
_CH = 128


def _make_edge_weight_kernel(e_pad, n_pad):
    chunks = e_pad // _CH
    cpt = chunks // _NS
    cpw = chunks // (_NC * _NS)
    stripe = n_pad // _NS

    mesh = plsc.VectorSubcoreMesh(core_axis_name="c", subcore_axis_name="s")

    @functools.partial(
        pl.kernel,
        out_type=jax.ShapeDtypeStruct((e_pad,), jnp.float32),
        mesh=mesh,
        scratch_types=[
            pltpu.VMEM((1, _CH), jnp.int32),
            pltpu.VMEM((_CH,), jnp.float32),
            pltpu.VMEM((stripe,), jnp.float32),
            pltpu.VMEM((n_pad,), jnp.float32),
            pltpu.VMEM((_CH,), jnp.int32),
            pltpu.VMEM((_CH,), jnp.float32),
            pltpu.VMEM((_CH,), jnp.float32),
            pltpu.VMEM_SHARED((n_pad,), jnp.float32),
        ],
        compiler_params=_sc_params(),
    )
    def ker(dst_hbm, ew_hbm, ew2_hbm, dstr, ones_v, zer_v, cntl, dch, ech, och,
            cnt_sh):
        c = lax.axis_index("c")
        s = lax.axis_index("s")

        for j in range(_CH // _L):
            ones_v[pl.ds(j * _L, _L)] = jnp.full((_L,), 1.0, jnp.float32)
        for j in range(stripe // _L):
            zer_v[pl.ds(j * _L, _L)] = jnp.zeros((_L,), jnp.float32)
        pltpu.sync_copy(zer_v, cnt_sh.at[pl.ds(s * stripe, stripe)])
        plsc.subcore_barrier()

        @pl.loop(0, cpt)
        def _(i):
            ch = s * cpt + i
            pltpu.sync_copy(dst_hbm.at[pl.ds(ch * _CH, _CH)], dstr.at[0])
            pltpu.sync_copy(ones_v, cnt_sh.at[dstr.at[0]], add=True)

        plsc.subcore_barrier()
        pltpu.sync_copy(cnt_sh, cntl)

        w = s * _NC + c

        @pl.loop(0, cpw)
        def _(i):
            ch = w * cpw + i
            pltpu.sync_copy(dst_hbm.at[pl.ds(ch * _CH, _CH)], dch)
            pltpu.sync_copy(ew_hbm.at[pl.ds(ch * _CH, _CH)], ech)
            for j in range(_CH // _L):
                sl = pl.ds(j * _L, _L)
                cv = plsc.load_gather(cntl, [dch[sl]])
                och[sl] = ech[sl] / jnp.maximum(cv, 1.0)
            pltpu.sync_copy(och, ew2_hbm.at[pl.ds(ch * _CH, _CH)])

    return ker


_AC = 64


def _make_agg_kernel(e_pad, n_pad, hh):
    chunks = e_pad // _AC
    cpt = chunks // _NS
    stripe = n_pad // _NS
    assert cpt % 2 == 0

    mesh = plsc.VectorSubcoreMesh(core_axis_name="c", subcore_axis_name="s")

    @functools.partial(
        pl.kernel,
        out_type=jax.ShapeDtypeStruct((_NC, n_pad, hh), jnp.float32),
        mesh=mesh,
        scratch_types=[
            pltpu.VMEM((cpt * _AC,), jnp.int32),
            pltpu.VMEM((cpt, _AC), jnp.int32),
            pltpu.VMEM((2, _AC), jnp.float32),
            pltpu.VMEM((2, _AC, hh), jnp.float32),
            pltpu.SemaphoreType.DMA,
            pltpu.SemaphoreType.DMA,
            pltpu.SemaphoreType.DMA,
            pltpu.SemaphoreType.DMA,
            pltpu.VMEM_SHARED((n_pad, hh), jnp.float32),
        ],
        compiler_params=_sc_params(),
    )
    def ker(y0_hbm, y1_hbm, src_hbm, dst2_hbm, ew2_hbm, agg_hbm,
            srcb, dstb, ewb, rows, sg0, sg1, ss0, ss1, agg_sh):
        c = lax.axis_index("c")
        s = lax.axis_index("s")
        sg = (sg0, sg1)
        ss = (ss0, ss1)

        def gather_start(i, b):
            idx = srcb.at[pl.ds(i * _AC, _AC)]
            pltpu.async_copy(ew2_hbm.at[pl.ds(s * cpt * _AC + i * _AC, _AC)],
                             ewb.at[b], sg[b])

            @pl.when(c == 0)
            def _():
                pltpu.async_copy(y0_hbm.at[idx], rows.at[b], sg[b])

            @pl.when(c != 0)
            def _():
                pltpu.async_copy(y1_hbm.at[idx], rows.at[b], sg[b])

        def gather_wait(b):
            pltpu.make_async_copy(y0_hbm.at[pl.ds(0, _AC)], rows.at[b],
                                  sg[b]).wait()
            pltpu.make_async_copy(ew2_hbm.at[pl.ds(0, _AC)], ewb.at[b],
                                  sg[b]).wait()

        def scale(i, b):
            @pl.loop(0, _AC // _L)
            def _(g):
                w16 = ewb[b, pl.ds(g * _L, _L)]
                for k in range(_L):
                    e = g * _L + k
                    wgt = w16[k]
                    for j in range(hh // _L):
                        sl = pl.ds(j * _L, _L)
                        rows[b, e, sl] = rows[b, e, sl] * wgt

        def scatter_start(i, b):
            pltpu.async_copy(rows.at[b], agg_sh.at[dstb.at[i]], ss[b],
                             add=True)

        def scatter_wait(b):
            pltpu.make_async_copy(rows.at[b], agg_sh.at[pl.ds(0, _AC)],
                                  ss[b]).wait()

        @pl.loop(0, _AC)
        def _(r):
            for j in range(hh // _L):
                rows[0, r, pl.ds(j * _L, _L)] = jnp.zeros((_L,), jnp.float32)

        @pl.loop(0, stripe // _AC)
        def _(t):
            pltpu.sync_copy(rows.at[0], agg_sh.at[pl.ds(s * stripe + t * _AC,
                                                        _AC)])

        base = s * cpt * _AC
        pltpu.sync_copy(src_hbm.at[pl.ds(base, cpt * _AC)], srcb)
        pltpu.sync_copy(dst2_hbm.at[pl.ds(s * cpt, cpt)], dstb)
        plsc.subcore_barrier()

        @pl.loop(0, cpt // 2)
        def _(ii):
            for b in range(2):
                i = ii * 2 + b
                ob = 1 - b

                @pl.when(i >= 2)
                def _():
                    scatter_wait(b)

                gather_start(i, b)

                @pl.when(i >= 1)
                def _():
                    gather_wait(ob)
                    scale(i - 1, ob)
                    scatter_start(i - 1, ob)

        gather_wait(1)
        scale(cpt - 1, 1)
        scatter_start(cpt - 1, 1)
        scatter_wait(0)
        scatter_wait(1)

        plsc.subcore_barrier()
        pltpu.sync_copy(agg_sh.at[pl.ds(s * stripe, stripe)],
                        agg_hbm.at[c].at[pl.ds(s * stripe, stripe)])

    return ker


def _prep_body(x_ref, w_ref, b_ref, o0_ref, o1_ref):
    acc = jnp.dot(x_ref[...], w_ref[...],
                  preferred_element_type=jnp.float32,
                  precision=lax.Precision.HIGHEST)
    acc = jnp.maximum(acc + b_ref[...], 0.0)
    o0_ref[...] = acc[:, :128]
    o1_ref[...] = acc[:, 128:]


def _prepare(x_p, wp, bp, bn):
    n_pad, d = x_p.shape
    h = wp.shape[1]
    return pl.pallas_call(
        _prep_body,
        grid=(n_pad // bn,),
        in_specs=[
            pl.BlockSpec((bn, d), lambda i: (i, 0)),
            pl.BlockSpec((d, h), lambda i: (0, 0)),
            pl.BlockSpec((1, h), lambda i: (0, 0)),
        ],
        out_specs=[pl.BlockSpec((bn, 128), lambda i: (i, 0)),
                   pl.BlockSpec((bn, 128), lambda i: (i, 0))],
        out_shape=[jax.ShapeDtypeStruct((n_pad, 128), jnp.float32),
                   jax.ShapeDtypeStruct((n_pad, 128), jnp.float32)],
    )(x_p, wp, bp.reshape(1, h))


def _upd_body(x_ref, a0_ref, a1_ref, wx_ref, w0_ref, w1_ref, b_ref, o_ref):
    acc = jnp.dot(x_ref[...], wx_ref[...],
                  preferred_element_type=jnp.float32,
                  precision=lax.Precision.HIGHEST)
    acc += jnp.dot(a0_ref[0], w0_ref[...],
                   preferred_element_type=jnp.float32,
                   precision=lax.Precision.HIGHEST)
    acc += jnp.dot(a1_ref[0], w1_ref[...],
                   preferred_element_type=jnp.float32,
                   precision=lax.Precision.HIGHEST)
    o_ref[...] = jnp.maximum(acc + b_ref[...], 0.0)


def _update(x_p, agg, wu, bu, bn):
    n_pad, d = x_p.shape
    h = wu.shape[1]
    hh = agg.shape[2]
    wx = wu[:d]
    w0 = wu[d:d + hh]
    w1 = wu[d + hh:]
    return pl.pallas_call(
        _upd_body,
        grid=(n_pad // bn,),
        in_specs=[
            pl.BlockSpec((bn, d), lambda i: (i, 0)),
            pl.BlockSpec((1, bn, hh), lambda i: (0, i, 0)),
            pl.BlockSpec((1, bn, hh), lambda i: (1, i, 0)),
            pl.BlockSpec((d, h), lambda i: (0, 0)),
            pl.BlockSpec((hh, h), lambda i: (0, 0)),
            pl.BlockSpec((hh, h), lambda i: (0, 0)),
            pl.BlockSpec((1, h), lambda i: (0, 0)),
        ],
        out_specs=pl.BlockSpec((bn, h), lambda i: (i, 0)),
        out_shape=jax.ShapeDtypeStruct((n_pad, h), jnp.float32),
    )(x_p, agg, agg, wx, w0, w1, bu.reshape(1, h))


def kernel(node_features, edge_index, edge_weights, Wp1, bp1, Wu1, bu1,
           Wp2, bp2, Wu2, bu2):
    n, d = node_features.shape
    h = Wp1.shape[1]
    e = edge_weights.shape[0]
    hh = h // _NC

    bn = 2048
    n_pad = -(-max(n + 1, 1) // bn) * bn
    e_pad = -(-e // (_CH * _NC * _NS)) * (_CH * _NC * _NS)

    src = edge_index[0]
    dst = edge_index[1]
    pad_e = e_pad - e
    src_p = jnp.concatenate([src, jnp.zeros((pad_e,), jnp.int32)])
    dst_p = jnp.concatenate([dst, jnp.full((pad_e,), n, jnp.int32)])
    ew_p = jnp.concatenate([edge_weights, jnp.zeros((pad_e,), jnp.float32)])
    dst2 = dst_p.reshape(e_pad // _AC, _AC)
    x_p = jnp.zeros((n_pad, d), jnp.float32).at[:n].set(node_features)

    ew2 = _make_edge_weight_kernel(e_pad, n_pad)(dst_p, ew_p)
    agg_fn = _make_agg_kernel(e_pad, n_pad, hh)

    y1a, y1b = _prepare(x_p, Wp1, bp1, bn)
    agg1 = agg_fn(y1a, y1b, src_p, dst2, ew2)
    e1 = _update(x_p, agg1, Wu1, bu1, bn)

    y2a, y2b = _prepare(e1, Wp2, bp2, bn)
    agg2 = agg_fn(y2a, y2b, src_p, dst2, ew2)
    e2 = _update(e1, agg2, Wu2, bu2, bn)

    return e2[:n]


# speedup vs baseline: 3.2609x; 1.0203x over previous
import dataclasses
import functools

import jax
import jax.numpy as jnp
from jax import lax
from jax.experimental import pallas as pl
from jax.experimental.pallas import tpu as pltpu
from jax.experimental.pallas import tpu_sc as plsc

def _sc_params():
    cp = pltpu.CompilerParams()
    if "needs_layout_passes" in pltpu.CompilerParams.__dataclass_fields__:
        cp = dataclasses.replace(cp, needs_layout_passes=False)
    return cp


_NC = 2
_NS = 16
_L = 16
_CH = 128


def _make_edge_weight_kernel(e_pad, n_pad):
    chunks = e_pad // _CH
    cpt = chunks // _NS
    cpw = chunks // (_NC * _NS)
    stripe = n_pad // _NS

    mesh = plsc.VectorSubcoreMesh(core_axis_name="c", subcore_axis_name="s")

    @functools.partial(
        pl.kernel,
        out_type=jax.ShapeDtypeStruct((e_pad,), jnp.float32),
        mesh=mesh,
        scratch_types=[
            pltpu.VMEM((1, _CH), jnp.int32),
            pltpu.VMEM((_CH,), jnp.float32),
            pltpu.VMEM((stripe,), jnp.float32),
            pltpu.VMEM((n_pad,), jnp.float32),
            pltpu.VMEM((_CH,), jnp.int32),
            pltpu.VMEM((_CH,), jnp.float32),
            pltpu.VMEM((_CH,), jnp.float32),
            pltpu.VMEM_SHARED((n_pad,), jnp.float32),
        ],
        compiler_params=_sc_params(),
    )
    def ker(dst_hbm, ew_hbm, ew2_hbm, dstr, ones_v, zer_v, cntl, dch, ech, och,
            cnt_sh):
        c = lax.axis_index("c")
        s = lax.axis_index("s")

        for j in range(_CH // _L):
            ones_v[pl.ds(j * _L, _L)] = jnp.full((_L,), 1.0, jnp.float32)
        for j in range(stripe // _L):
            zer_v[pl.ds(j * _L, _L)] = jnp.zeros((_L,), jnp.float32)
        pltpu.sync_copy(zer_v, cnt_sh.at[pl.ds(s * stripe, stripe)])
        plsc.subcore_barrier()

        @pl.loop(0, cpt)
        def _(i):
            ch = s * cpt + i
            pltpu.sync_copy(dst_hbm.at[pl.ds(ch * _CH, _CH)], dstr.at[0])
            pltpu.sync_copy(ones_v, cnt_sh.at[dstr.at[0]], add=True)

        plsc.subcore_barrier()
        pltpu.sync_copy(cnt_sh, cntl)

        w = s * _NC + c

        @pl.loop(0, cpw)
        def _(i):
            ch = w * cpw + i
            pltpu.sync_copy(dst_hbm.at[pl.ds(ch * _CH, _CH)], dch)
            pltpu.sync_copy(ew_hbm.at[pl.ds(ch * _CH, _CH)], ech)
            for j in range(_CH // _L):
                sl = pl.ds(j * _L, _L)
                cv = plsc.load_gather(cntl, [dch[sl]])
                och[sl] = ech[sl] / jnp.maximum(cv, 1.0)
            pltpu.sync_copy(och, ew2_hbm.at[pl.ds(ch * _CH, _CH)])

    return ker


_AC = 128
_NI = 4
_NR = 2


def _make_agg_kernel(e_pad, n_pad, hh):
    chunks = e_pad // _AC
    cpt = chunks // _NS
    stripe = n_pad // _NS
    assert cpt % _NI == 0

    mesh = plsc.VectorSubcoreMesh(core_axis_name="c", subcore_axis_name="s")

    @functools.partial(
        pl.kernel,
        out_type=jax.ShapeDtypeStruct((_NC, n_pad, hh), jnp.float32),
        mesh=mesh,
        scratch_types=[
            pltpu.VMEM((_NI, _AC), jnp.int32),
            pltpu.VMEM((_NI, _AC), jnp.int32),
            pltpu.VMEM((_NI, _AC), jnp.float32),
            pltpu.VMEM((_NR, _AC, hh), jnp.float32),
            [pltpu.SemaphoreType.DMA] * _NI,
            [pltpu.SemaphoreType.DMA] * _NR,
            [pltpu.SemaphoreType.DMA] * _NR,
            pltpu.VMEM_SHARED((n_pad, hh), jnp.float32),
        ],
        compiler_params=_sc_params(),
    )
    def ker(y0_hbm, y1_hbm, src_hbm, dst2_hbm, ew2_hbm, agg_hbm,
            srcb, dstb, ewb, rows, si, sg, ss, agg_sh):
        c = lax.axis_index("c")
        s = lax.axis_index("s")
        base_ch = s * cpt

        def idx_start(i, q):
            off = (base_ch + i) * _AC
            pltpu.async_copy(src_hbm.at[pl.ds(off, _AC)], srcb.at[q], si[q])
            pltpu.async_copy(dst2_hbm.at[base_ch + i], dstb.at[q], si[q])
            pltpu.async_copy(ew2_hbm.at[pl.ds(off, _AC)], ewb.at[q], si[q])

        def idx_wait(q):
            pltpu.make_async_copy(src_hbm.at[pl.ds(0, _AC)], srcb.at[q],
                                  si[q]).wait()
            pltpu.make_async_copy(src_hbm.at[pl.ds(0, _AC)], dstb.at[q],
                                  si[q]).wait()
            pltpu.make_async_copy(ew2_hbm.at[pl.ds(0, _AC)], ewb.at[q],
                                  si[q]).wait()

        def gather_start(q, b):
            idx = srcb.at[q]

            @pl.when(c == 0)
            def _():
                pltpu.async_copy(y0_hbm.at[idx], rows.at[b], sg[b])

            @pl.when(c != 0)
            def _():
                pltpu.async_copy(y1_hbm.at[idx], rows.at[b], sg[b])

        def gather_wait(b):
            pltpu.make_async_copy(y0_hbm.at[pl.ds(0, _AC)], rows.at[b],
                                  sg[b]).wait()

        def scale(q, b):
            @pl.loop(0, _AC // _L)
            def _(g):
                w16 = ewb[q, pl.ds(g * _L, _L)]
                for k in range(_L):
                    e = g * _L + k
                    wgt = w16[k]
                    for j in range(hh // _L):
                        sl = pl.ds(j * _L, _L)
                        rows[b, e, sl] = rows[b, e, sl] * wgt

        def scatter_start(q, b):
            pltpu.async_copy(rows.at[b], agg_sh.at[dstb.at[q]], ss[b],
                             add=True)

        def scatter_wait(b):
            pltpu.make_async_copy(rows.at[b], agg_sh.at[pl.ds(0, _AC)],
                                  ss[b]).wait()

        @pl.loop(0, _AC)
        def _(r):
            for j in range(hh // _L):
                rows[0, r, pl.ds(j * _L, _L)] = jnp.zeros((_L,), jnp.float32)

        @pl.loop(0, stripe // _AC)
        def _(t):
            pltpu.sync_copy(rows.at[0], agg_sh.at[pl.ds(s * stripe + t * _AC,
                                                        _AC)])
        plsc.subcore_barrier()

        idx_start(0, 0)
        idx_start(1, 1)

        @pl.loop(0, cpt // _NI)
        def _(ii):
            for b4 in range(_NI):
                i = ii * _NI + b4
                q = b4
                b = b4 % _NR
                pq = (b4 - 1) % _NI
                pb = (b4 - 1) % _NR

                @pl.when(i >= _NR)
                def _():
                    scatter_wait(b)

                idx_wait(q)
                gather_start(q, b)

                @pl.when(i + 2 < cpt)
                def _():
                    idx_start(i + 2, (b4 + 2) % _NI)

                @pl.when(i >= 1)
                def _():
                    gather_wait(pb)
                    scale(pq, pb)
                    scatter_start(pq, pb)

        lb = (cpt - 1) % _NR
        lq = (cpt - 1) % _NI
        gather_wait(lb)
        scale(lq, lb)
        scatter_start(lq, lb)
        scatter_wait(0)
        scatter_wait(1)

        plsc.subcore_barrier()
        pltpu.sync_copy(agg_sh.at[pl.ds(s * stripe, stripe)],
                        agg_hbm.at[c].at[pl.ds(s * stripe, stripe)])

    return ker


def _prep_body(x_ref, w_ref, b_ref, o0_ref, o1_ref):
    acc = jnp.dot(x_ref[...], w_ref[...],
                  preferred_element_type=jnp.float32,
                  precision=lax.Precision.HIGHEST)
    acc = jnp.maximum(acc + b_ref[...], 0.0)
    o0_ref[...] = acc[:, :128]
    o1_ref[...] = acc[:, 128:]


def _prepare(x_p, wp, bp, bn):
    n_pad, d = x_p.shape
    h = wp.shape[1]
    return pl.pallas_call(
        _prep_body,
        grid=(n_pad // bn,),
        in_specs=[
            pl.BlockSpec((bn, d), lambda i: (i, 0)),
            pl.BlockSpec((d, h), lambda i: (0, 0)),
            pl.BlockSpec((1, h), lambda i: (0, 0)),
        ],
        out_specs=[pl.BlockSpec((bn, 128), lambda i: (i, 0)),
                   pl.BlockSpec((bn, 128), lambda i: (i, 0))],
        out_shape=[jax.ShapeDtypeStruct((n_pad, 128), jnp.float32),
                   jax.ShapeDtypeStruct((n_pad, 128), jnp.float32)],
    )(x_p, wp, bp.reshape(1, h))


def _upd_body(x_ref, a0_ref, a1_ref, wx_ref, w0_ref, w1_ref, b_ref, o_ref):
    acc = jnp.dot(x_ref[...], wx_ref[...],
                  preferred_element_type=jnp.float32,
                  precision=lax.Precision.HIGHEST)
    acc += jnp.dot(a0_ref[0], w0_ref[...],
                   preferred_element_type=jnp.float32,
                   precision=lax.Precision.HIGHEST)
    acc += jnp.dot(a1_ref[0], w1_ref[...],
                   preferred_element_type=jnp.float32,
                   precision=lax.Precision.HIGHEST)
    o_ref[...] = jnp.maximum(acc + b_ref[...], 0.0)


def _update(x_p, agg, wu, bu, bn):
    n_pad, d = x_p.shape
    h = wu.shape[1]
    hh = agg.shape[2]
    wx = wu[:d]
    w0 = wu[d:d + hh]
    w1 = wu[d + hh:]
    return pl.pallas_call(
        _upd_body,
        grid=(n_pad // bn,),
        in_specs=[
            pl.BlockSpec((bn, d), lambda i: (i, 0)),
            pl.BlockSpec((1, bn, hh), lambda i: (0, i, 0)),
            pl.BlockSpec((1, bn, hh), lambda i: (1, i, 0)),
            pl.BlockSpec((d, h), lambda i: (0, 0)),
            pl.BlockSpec((hh, h), lambda i: (0, 0)),
            pl.BlockSpec((hh, h), lambda i: (0, 0)),
            pl.BlockSpec((1, h), lambda i: (0, 0)),
        ],
        out_specs=pl.BlockSpec((bn, h), lambda i: (i, 0)),
        out_shape=jax.ShapeDtypeStruct((n_pad, h), jnp.float32),
    )(x_p, agg, agg, wx, w0, w1, bu.reshape(1, h))


def kernel(node_features, edge_index, edge_weights, Wp1, bp1, Wu1, bu1,
           Wp2, bp2, Wu2, bu2):
    n, d = node_features.shape
    h = Wp1.shape[1]
    e = edge_weights.shape[0]
    hh = h // _NC

    bn = 2048
    n_pad = -(-max(n + 1, 1) // bn) * bn
    e_pad = -(-e // (_CH * _NC * _NS)) * (_CH * _NC * _NS)

    src = edge_index[0]
    dst = edge_index[1]
    pad_e = e_pad - e
    src_p = jnp.concatenate([src, jnp.zeros((pad_e,), jnp.int32)])
    dst_p = jnp.concatenate([dst, jnp.full((pad_e,), n, jnp.int32)])
    ew_p = jnp.concatenate([edge_weights, jnp.zeros((pad_e,), jnp.float32)])
    dst2 = dst_p.reshape(e_pad // _AC, _AC)
    x_p = jnp.zeros((n_pad, d), jnp.float32).at[:n].set(node_features)

    ew2 = _make_edge_weight_kernel(e_pad, n_pad)(dst_p, ew_p)
    agg_fn = _make_agg_kernel(e_pad, n_pad, hh)

    y1a, y1b = _prepare(x_p, Wp1, bp1, bn)
    agg1 = agg_fn(y1a, y1b, src_p, dst2, ew2)
    e1 = _update(x_p, agg1, Wu1, bu1, bn)

    y2a, y2b = _prepare(e1, Wp2, bp2, bn)
    agg2 = agg_fn(y2a, y2b, src_p, dst2, ew2)
    e2 = _update(e1, agg2, Wu2, bu2, bn)

    return e2[:n]
